# baseline (speedup 1.0000x reference)
import functools

import jax
import jax.numpy as jnp
from jax import lax
from jax.experimental import pallas as pl
from jax.experimental.pallas import tpu as pltpu
from jax.experimental.pallas import tpu_sc as plsc

D_MODEL = 64
SCALE = 8.0
NUM_CORES = 2
NUM_SUBCORES = 16
NUM_WORKERS = NUM_CORES * NUM_SUBCORES
SUB = 128
OUTER = 1024
INNER = 512
LANES = 16


@functools.partial(jax.jit, static_argnames=("n",))
def _sc_embed(x2d, table2, n):
    per_w = n // NUM_WORKERS
    n_outer = per_w // OUTER
    mesh = plsc.VectorSubcoreMesh(
        core_axis_name="c",
        subcore_axis_name="s",
        num_cores=NUM_CORES,
        num_subcores=NUM_SUBCORES,
    )

    @functools.partial(
        pl.kernel,
        mesh=mesh,
        out_type=jax.ShapeDtypeStruct((n // 2, 2 * D_MODEL), jnp.float32),
        scratch_types=[
            pltpu.VMEM((OUTER // SUB, SUB), jnp.int32),
            pltpu.VMEM((OUTER // SUB, SUB), jnp.int32),
            pltpu.VMEM((OUTER // SUB, SUB), jnp.int32),
            pltpu.VMEM((INNER, 2 * D_MODEL), jnp.float32),
            pltpu.VMEM((INNER // 2, 2 * D_MODEL), jnp.float32),
            pltpu.SemaphoreType.DMA,
        ],
    )
    def body(x_hbm, tab_hbm, out_hbm, xb, pairb, halfb, rows_v, outb, sem):
        wid = lax.axis_index("s") * NUM_CORES + lax.axis_index("c")
        base = wid * per_w
        base_row = base // SUB

        def outer_body(g, _):
            off = pl.multiple_of(base + g * OUTER, OUTER)
            row0 = pl.multiple_of(base_row + g * (OUTER // SUB), OUTER // SUB)
            pltpu.sync_copy(x_hbm.at[pl.ds(row0, OUTER // SUB)], xb)

            for r in range(OUTER // SUB):
                for c in range(SUB // LANES):
                    sl = pl.ds(c * LANES, LANES)
                    tok = xb[r, sl]
                    pairb[r, sl] = lax.shift_right_logical(tok, 1)
                    halfb[r, sl] = lax.shift_left(tok & 1, 6)

            for sub in range(OUTER // INNER):
                copies = []
                for j in range(INNER // SUB):
                    jrow = sub * (INNER // SUB) + j
                    copies.append(
                        pltpu.async_copy(
                            tab_hbm.at[pairb.at[jrow]],
                            rows_v.at[pl.ds(j * SUB, SUB)],
                            sem,
                        )
                    )
                for cp in copies:
                    cp.wait()

                def scale_body(g2, _):
                    p0 = sub * INNER + g2 * LANES
                    hv = halfb[p0 >> 7, pl.ds(p0 & (SUB - 1), LANES)]
                    for i in range(LANES):
                        r = g2 * LANES + i
                        ro = g2 * (LANES // 2) + i // 2
                        h = hv[i]
                        for c in range(D_MODEL // LANES):
                            src = pl.ds(h + c * LANES, LANES)
                            dst = pl.ds(
                                (i % 2) * D_MODEL + c * LANES, LANES
                            )
                            outb[ro, dst] = rows_v[r, src] * SCALE
                    return ()

                lax.fori_loop(0, INNER // LANES, scale_body, ())
                pltpu.sync_copy(
                    outb,
                    out_hbm.at[
                        pl.ds(
                            pl.multiple_of(
                                (off + sub * INNER) // 2, INNER // 2
                            ),
                            INNER // 2,
                        )
                    ],
                )
            return ()

        lax.fori_loop(0, n_outer, outer_body, ())

    return body(x2d, table2)


def kernel(x, table):
    b, s = x.shape
    n = b * s
    x2d = x.reshape(n // SUB, SUB).astype(jnp.int32)
    table2 = table.reshape(-1, 2 * D_MODEL)
    out = _sc_embed(x2d, table2, n)
    return out.reshape(b, s, D_MODEL)


# speedup vs baseline: 1.4898x; 1.4898x over previous
import functools

import jax
import jax.numpy as jnp
from jax import lax
from jax.experimental import pallas as pl
from jax.experimental.pallas import tpu as pltpu
from jax.experimental.pallas import tpu_sc as plsc

D_MODEL = 64
SCALE = 8.0
NUM_CORES = 2
NUM_SUBCORES = 16
NUM_WORKERS = NUM_CORES * NUM_SUBCORES
SUB = 128
K = 4
CHUNK = SUB * K
LANES = 16


@functools.partial(jax.jit, static_argnames=("n",))
def _sc_embed(x2d, table, n):
    per_w = n // NUM_WORKERS
    n_chunks = per_w // CHUNK
    n_pairs = n_chunks // 2
    idx_rows = per_w // SUB
    mesh = plsc.VectorSubcoreMesh(
        core_axis_name="c",
        subcore_axis_name="s",
        num_cores=NUM_CORES,
        num_subcores=NUM_SUBCORES,
    )

    @functools.partial(
        pl.kernel,
        mesh=mesh,
        out_type=jax.ShapeDtypeStruct((n, D_MODEL), jnp.float32),
        scratch_types=[
            pltpu.VMEM((idx_rows, SUB), jnp.int32),
            pltpu.VMEM((CHUNK, D_MODEL), jnp.float32),
            pltpu.VMEM((CHUNK, D_MODEL), jnp.float32),
            pltpu.SemaphoreType.DMA,
            pltpu.SemaphoreType.DMA,
            pltpu.SemaphoreType.DMA,
            pltpu.SemaphoreType.DMA,
        ],
        compiler_params=pltpu.CompilerParams(use_tc_tiling_on_sc=False),
    )
    def body(x_hbm, tab_hbm, out_hbm, idxb, buf0, buf1, sg0, sg1, so0, so1):
        wid = lax.axis_index("s") * NUM_CORES + lax.axis_index("c")
        base = wid * per_w
        base_row = base // SUB
        bufs = (buf0, buf1)
        gsems = (sg0, sg1)
        osems = (so0, so1)

        pltpu.sync_copy(x_hbm.at[pl.ds(base_row, idx_rows)], idxb)

        def fire_gathers(g, buf, sem):
            for j in range(K):
                pltpu.async_copy(
                    tab_hbm.at[idxb.at[g * K + j]],
                    buf.at[pl.ds(j * SUB, SUB)],
                    sem,
                )

        def wait_gathers(g, buf, sem):
            for j in range(K):
                pltpu.make_async_copy(
                    tab_hbm.at[idxb.at[g * K + j]],
                    buf.at[pl.ds(j * SUB, SUB)],
                    sem,
                ).wait()

        def out_slice(g):
            return out_hbm.at[
                pl.ds(pl.multiple_of(base + g * CHUNK, CHUNK), CHUNK)
            ]

        fire_gathers(0, buf0, sg0)

        def pair_body(t, _):
            for phase in range(2):
                g = 2 * t + phase
                buf = bufs[phase]
                other = bufs[1 - phase]
                wait_gathers(g, buf, gsems[phase])

                if phase == 0:
                    @pl.when(t >= 1)
                    def _():
                        pltpu.make_async_copy(
                            other, out_slice(g - 1), osems[1 - phase]
                        ).wait()
                        fire_gathers(g + 1, other, gsems[1 - phase])

                    @pl.when(t == 0)
                    def _():
                        fire_gathers(g + 1, other, gsems[1 - phase])
                else:
                    @pl.when(t < n_pairs - 1)
                    def _():
                        pltpu.make_async_copy(
                            other, out_slice(g - 1), osems[1 - phase]
                        ).wait()
                        fire_gathers(g + 1, other, gsems[1 - phase])

                def scale_body(r, _):
                    for c in range(D_MODEL // LANES):
                        sl = pl.ds(c * LANES, LANES)
                        buf[r, sl] = buf[r, sl] * SCALE
                    return ()

                lax.fori_loop(0, CHUNK, scale_body, (), unroll=2)
                pltpu.async_copy(buf, out_slice(g), osems[phase])
            return ()

        lax.fori_loop(0, n_pairs, pair_body, ())
        pltpu.make_async_copy(
            buf0, out_slice(n_chunks - 2), so0
        ).wait()
        pltpu.make_async_copy(
            buf1, out_slice(n_chunks - 1), so1
        ).wait()

    return body(x2d, table)


def kernel(x, table):
    b, s = x.shape
    n = b * s
    x2d = x.reshape(n // SUB, SUB).astype(jnp.int32)
    out = _sc_embed(x2d, table, n)
    return out.reshape(b, s, D_MODEL)
